# Initial kernel scaffold; baseline (speedup 1.0000x reference)
import jax
import jax.numpy as jnp
from jax.experimental import pallas as pl


def kernel(tokenized_prompts, token_embedding, ctx):
    raise NotImplementedError("write your pallas kernel here")


# speedup vs baseline: 3.9681x; 3.9681x over previous
import functools

import jax
import jax.numpy as jnp
from jax import lax
from jax.experimental import pallas as pl
from jax.experimental.pallas import tpu as pltpu
from jax.experimental.pallas import tpu_sc as plsc

N_CLS_K = 1000
SEQ_K = 77
N_CTX_K = 16
CTX_DIM_K = 512
HEAD = 24
TAIL = SEQ_K - HEAD
IDX_STRIDE = 72

NC, NS = 2, 16
NW = NC * NS
BASE_CLS = N_CLS_K // NW
EXTRA = N_CLS_K - BASE_CLS * NW
MAX_CLS = BASE_CLS + 1


def _make_kernel():
    mesh = plsc.VectorSubcoreMesh(core_axis_name="c", subcore_axis_name="s")

    @functools.partial(
        pl.kernel,
        mesh=mesh,
        out_type=jax.ShapeDtypeStruct((N_CLS_K, SEQ_K, CTX_DIM_K), jnp.float32),
        scratch_types=[
            pltpu.VMEM((MAX_CLS * IDX_STRIDE,), jnp.int32),
            pltpu.VMEM((HEAD, CTX_DIM_K), jnp.float32),
            pltpu.VMEM((16, CTX_DIM_K), jnp.float32),
            pltpu.VMEM((48, CTX_DIM_K), jnp.float32),
            pltpu.VMEM((5, CTX_DIM_K), jnp.float32),
            pltpu.SemaphoreType.DMA,
        ],
    )
    def sc_kernel(
        idx_hbm, table_hbm, head_hbm, out_hbm, idx_v, head_v, g16_v, tail_v, e5_v, sem
    ):
        wid = lax.axis_index("s") * NC + lax.axis_index("c")
        n_cls = jnp.where(wid < EXTRA, BASE_CLS + 1, BASE_CLS)
        start = BASE_CLS * wid + jnp.minimum(wid, EXTRA)

        pltpu.sync_copy(
            idx_hbm.at[pl.ds(pl.multiple_of(start * IDX_STRIDE, 8), MAX_CLS * IDX_STRIDE)],
            idx_v,
        )
        pltpu.sync_copy(head_hbm, head_v)

        def step(k, carry):
            c = start + k
            base = pl.multiple_of(k * IDX_STRIDE, 8)
            pltpu.async_copy(
                table_hbm.at[idx_v.at[pl.ds(base, 16)]], g16_v, sem
            ).wait()
            for r in range(13):
                for j in range(CTX_DIM_K // 16):
                    v = g16_v[r, pl.ds(j * 16, 16)]
                    if r == 0:
                        head_v[0, pl.ds(j * 16, 16)] = v
                    elif r < 8:
                        head_v[16 + r, pl.ds(j * 16, 16)] = v
                    else:
                        e5_v[r - 8, pl.ds(j * 16, 16)] = v
            pltpu.async_copy(
                table_hbm.at[idx_v.at[pl.ds(base + 16, 48)]],
                tail_v,
                sem,
            ).wait()
            pltpu.sync_copy(head_v, out_hbm.at[c, pl.ds(0, HEAD)])
            pltpu.sync_copy(tail_v, out_hbm.at[c, pl.ds(HEAD, 48)])
            pltpu.sync_copy(e5_v, out_hbm.at[c, pl.ds(72, 5)])
            return carry

        lax.fori_loop(0, n_cls, step, 0)

    return sc_kernel


_SC_KERNEL = _make_kernel()


@jax.jit
def kernel(tokenized_prompts, token_embedding, ctx):
    prefix = tokenized_prompts[:, :1]
    suffix = tokenized_prompts[:, 1 + N_CTX_K :]
    pad = jnp.broadcast_to(prefix, (N_CLS_K, 8))
    idx = jnp.concatenate(
        [prefix, suffix[:, :7], suffix[:, 55:60], pad[:, :3],
         suffix[:, 7:55], pad], axis=1
    ).reshape(-1)
    idx = jnp.concatenate([idx, jnp.zeros((IDX_STRIDE,), jnp.int32)])
    head = jnp.concatenate(
        [jnp.zeros((1, CTX_DIM_K), jnp.float32), ctx,
         jnp.zeros((7, CTX_DIM_K), jnp.float32)], axis=0
    )
    return _SC_KERNEL(idx, token_embedding, head)


# speedup vs baseline: 4.8552x; 1.2236x over previous
import functools

import jax
import jax.numpy as jnp
from jax import lax
from jax.experimental import pallas as pl
from jax.experimental.pallas import tpu as pltpu
from jax.experimental.pallas import tpu_sc as plsc

N_CLS_K = 1000
SEQ_K = 77
N_CTX_K = 16
CTX_DIM_K = 512
HEAD = 24
IDX_STRIDE = 72

NC, NS = 2, 16
NW = NC * NS
BASE_CLS = N_CLS_K // NW
EXTRA = N_CLS_K - BASE_CLS * NW
MAX_CLS = BASE_CLS + 1


def _make_kernel():
    mesh = plsc.VectorSubcoreMesh(core_axis_name="c", subcore_axis_name="s")

    @functools.partial(
        pl.kernel,
        mesh=mesh,
        out_type=jax.ShapeDtypeStruct((N_CLS_K, SEQ_K, CTX_DIM_K), jnp.float32),
        scratch_types=[
            pltpu.VMEM((MAX_CLS * IDX_STRIDE,), jnp.int32),
            pltpu.VMEM((HEAD, CTX_DIM_K), jnp.float32),
            pltpu.VMEM((HEAD, CTX_DIM_K), jnp.float32),
            pltpu.VMEM((16, CTX_DIM_K), jnp.float32),
            pltpu.VMEM((16, CTX_DIM_K), jnp.float32),
            pltpu.VMEM((48, CTX_DIM_K), jnp.float32),
            pltpu.VMEM((48, CTX_DIM_K), jnp.float32),
            pltpu.VMEM((5, CTX_DIM_K), jnp.float32),
            pltpu.VMEM((5, CTX_DIM_K), jnp.float32),
            pltpu.SemaphoreType.DMA,
            pltpu.SemaphoreType.DMA,
            pltpu.SemaphoreType.DMA,
            pltpu.SemaphoreType.DMA,
        ],
    )
    def sc_kernel(
        idx_hbm, table_hbm, head_hbm, out_hbm,
        idx_v, h0, h1, g0, g1, t0, t1, e0, e1, sg0, sg1, sw0, sw1,
    ):
        heads, g16s, tails, e5s = (h0, h1), (g0, g1), (t0, t1), (e0, e1)
        sgs, sws = (sg0, sg1), (sw0, sw1)
        wid = lax.axis_index("s") * NC + lax.axis_index("c")
        n_cls = jnp.where(wid < EXTRA, BASE_CLS + 1, BASE_CLS)
        start = BASE_CLS * wid + jnp.minimum(wid, EXTRA)

        pltpu.sync_copy(
            idx_hbm.at[pl.ds(pl.multiple_of(start * IDX_STRIDE, 8), MAX_CLS * IDX_STRIDE)],
            idx_v,
        )
        pltpu.sync_copy(head_hbm, h0)
        pltpu.sync_copy(head_hbm, h1)

        def issue_gathers(k, p):
            base = pl.multiple_of(lax.rem(k, n_cls) * IDX_STRIDE, 8)
            pltpu.async_copy(
                table_hbm.at[idx_v.at[pl.ds(base, 16)]], g16s[p], sgs[p]
            )
            pltpu.async_copy(
                table_hbm.at[idx_v.at[pl.ds(base + 16, 48)]], tails[p], sgs[p]
            )

        def drain_writes(p, c):
            pltpu.make_async_copy(heads[p], out_hbm.at[c, pl.ds(0, HEAD)], sws[p]).wait()
            pltpu.make_async_copy(tails[p], out_hbm.at[c, pl.ds(HEAD, 48)], sws[p]).wait()
            pltpu.make_async_copy(e5s[p], out_hbm.at[c, pl.ds(72, 5)], sws[p]).wait()

        issue_gathers(0, 0)

        def pair(kk, carry):
            for b in (0, 1):
                p, q = b, 1 - b
                k = kk * 2 + b
                c = start + lax.rem(k, n_cls)
                pltpu.make_async_copy(
                    table_hbm.at[pl.ds(0, 16)], g16s[p], sgs[p]
                ).wait()
                pltpu.make_async_copy(
                    table_hbm.at[pl.ds(0, 48)], tails[p], sgs[p]
                ).wait()

                @pl.when(k < MAX_CLS - 1)
                def _():
                    @pl.when(k >= 1)
                    def _():
                        drain_writes(q, c)
                    issue_gathers(k + 1, q)

                for r in range(13):
                    for j in range(CTX_DIM_K // 16):
                        v = g16s[p][r, pl.ds(j * 16, 16)]
                        if r == 0:
                            heads[p][0, pl.ds(j * 16, 16)] = v
                        elif r < 8:
                            heads[p][16 + r, pl.ds(j * 16, 16)] = v
                        else:
                            e5s[p][r - 8, pl.ds(j * 16, 16)] = v

                pltpu.async_copy(heads[p], out_hbm.at[c, pl.ds(0, HEAD)], sws[p])
                pltpu.async_copy(tails[p], out_hbm.at[c, pl.ds(HEAD, 48)], sws[p])
                pltpu.async_copy(e5s[p], out_hbm.at[c, pl.ds(72, 5)], sws[p])
            return carry

        lax.fori_loop(0, MAX_CLS // 2, pair, 0)

        drain_writes(0, start)
        drain_writes(1, start)

    return sc_kernel


_SC_KERNEL = _make_kernel()


@jax.jit
def kernel(tokenized_prompts, token_embedding, ctx):
    prefix = tokenized_prompts[:, :1]
    suffix = tokenized_prompts[:, 1 + N_CTX_K :]
    pad = jnp.broadcast_to(prefix, (N_CLS_K, 8))
    idx = jnp.concatenate(
        [prefix, suffix[:, :7], suffix[:, 55:60], pad[:, :3],
         suffix[:, 7:55], pad], axis=1
    ).reshape(-1)
    idx = jnp.concatenate([idx, jnp.zeros((IDX_STRIDE,), jnp.int32)])
    head = jnp.concatenate(
        [jnp.zeros((1, CTX_DIM_K), jnp.float32), ctx,
         jnp.zeros((7, CTX_DIM_K), jnp.float32)], axis=0
    )
    return _SC_KERNEL(idx, token_embedding, head)


# speedup vs baseline: 7.7506x; 1.5963x over previous
import functools

import jax
import jax.numpy as jnp
from jax import lax
from jax.experimental import pallas as pl
from jax.experimental.pallas import tpu as pltpu
from jax.experimental.pallas import tpu_sc as plsc

N_CLS_K = 1000
SEQ_K = 77
N_CTX_K = 16
CTX_DIM_K = 512
CHUNK = 64
N_CHUNKS = 16
LAST_START = N_CLS_K - CHUNK
N_TASKS = SEQ_K * N_CHUNKS

NC, NS = 2, 16
NW = NC * NS
BASE_T = N_TASKS // NW
EXTRA_T = N_TASKS - BASE_T * NW
ROUNDS = BASE_T + 1
assert ROUNDS % 3 == 0


def _make_kernel():
    mesh = plsc.VectorSubcoreMesh(core_axis_name="c", subcore_axis_name="s")

    @functools.partial(
        pl.kernel,
        mesh=mesh,
        out_type=jax.ShapeDtypeStruct((SEQ_K, N_CLS_K, CTX_DIM_K), jnp.float32),
        scratch_types=[
            pltpu.VMEM((CHUNK, CTX_DIM_K), jnp.float32),
            pltpu.VMEM((CHUNK, CTX_DIM_K), jnp.float32),
            pltpu.VMEM((CHUNK, CTX_DIM_K), jnp.float32),
            pltpu.VMEM((CHUNK,), jnp.int32),
            pltpu.VMEM((CHUNK,), jnp.int32),
            pltpu.VMEM((CHUNK,), jnp.int32),
            pltpu.SemaphoreType.DMA,
            pltpu.SemaphoreType.DMA,
            pltpu.SemaphoreType.DMA,
            pltpu.SemaphoreType.DMA,
            pltpu.SemaphoreType.DMA,
            pltpu.SemaphoreType.DMA,
            pltpu.SemaphoreType.DMA,
            pltpu.SemaphoreType.DMA,
            pltpu.SemaphoreType.DMA,
        ],
    )
    def sc_kernel(
        idx_hbm, table_hbm, ctxb_hbm, out_hbm,
        buf0, buf1, buf2, ib0, ib1, ib2,
        si0, si1, si2, sin0, sin1, sin2, sw0, sw1, sw2,
    ):
        bufs, ibufs = (buf0, buf1, buf2), (ib0, ib1, ib2)
        sidx, sin, sw = (si0, si1, si2), (sin0, sin1, sin2), (sw0, sw1, sw2)
        wid = lax.axis_index("s") * NC + lax.axis_index("c")
        n_t = jnp.where(wid < EXTRA_T, BASE_T + 1, BASE_T)

        def params(t_local):
            t = wid + NW * lax.rem(t_local, n_t)
            s = t // N_CHUNKS
            ch = lax.rem(t, N_CHUNKS)
            c0 = pl.multiple_of(
                jnp.where(ch == N_CHUNKS - 1, LAST_START, ch * CHUNK), 8
            )
            is_ctx = jnp.logical_and(s >= 1, s < 1 + N_CTX_K)
            row = jnp.where(is_ctx, 0, jnp.where(s == 0, 0, s - N_CTX_K))
            ioff = pl.multiple_of(row * N_CLS_K + c0, 8)
            return s, c0, is_ctx, ioff

        def stage_idx(t_local, b):
            _, _, _, ioff = params(t_local)
            pltpu.async_copy(
                idx_hbm.at[pl.ds(ioff, CHUNK)], ibufs[b], sidx[b]
            )

        def wait_idx(b):
            pltpu.make_async_copy(
                idx_hbm.at[pl.ds(0, CHUNK)], ibufs[b], sidx[b]
            ).wait()

        def issue_main(t_local, b):
            s, _, is_ctx, _ = params(t_local)

            @pl.when(is_ctx)
            def _():
                pltpu.async_copy(ctxb_hbm.at[s - 1], bufs[b], sin[b])

            @pl.when(jnp.logical_not(is_ctx))
            def _():
                pltpu.async_copy(table_hbm.at[ibufs[b]], bufs[b], sin[b])

        def wait_main(b):
            pltpu.make_async_copy(ctxb_hbm.at[0], bufs[b], sin[b]).wait()

        def drain_write(b):
            pltpu.make_async_copy(
                bufs[b], out_hbm.at[0, pl.ds(0, CHUNK)], sw[b]
            ).wait()

        stage_idx(0, 0)
        stage_idx(1, 1)
        wait_idx(0)
        issue_main(0, 0)

        def tripple(tt, carry):
            for b in (0, 1, 2):
                t = tt * 3 + b
                b1, b2 = (b + 1) % 3, (b + 2) % 3
                s, c0, _, _ = params(t)
                wait_main(b)
                pltpu.async_copy(
                    bufs[b], out_hbm.at[s, pl.ds(c0, CHUNK)], sw[b]
                )

                @pl.when(t + 2 < ROUNDS)
                def _():
                    stage_idx(t + 2, b2)

                @pl.when(t + 1 < ROUNDS)
                def _():
                    wait_idx(b1)

                    @pl.when(t >= 2)
                    def _():
                        drain_write(b1)

                    issue_main(t + 1, b1)
            return carry

        lax.fori_loop(0, ROUNDS // 3, tripple, 0)
        drain_write(0)
        drain_write(1)
        drain_write(2)

    return sc_kernel


_SC_KERNEL = _make_kernel()


@jax.jit
def kernel(tokenized_prompts, token_embedding, ctx):
    cols = jnp.concatenate(
        [tokenized_prompts[:, :1], tokenized_prompts[:, 1 + N_CTX_K :]], axis=1
    )
    idx = cols.T.reshape(-1)
    ctxb = jnp.broadcast_to(ctx[:, None, :], (N_CTX_K, CHUNK, CTX_DIM_K))
    out_t = _SC_KERNEL(idx, token_embedding, ctxb)
    return jnp.transpose(out_t, (1, 0, 2))


# speedup vs baseline: 8.7747x; 1.1321x over previous
import functools

import jax
import jax.numpy as jnp
from jax import lax
from jax.experimental import pallas as pl
from jax.experimental.pallas import tpu as pltpu
from jax.experimental.pallas import tpu_sc as plsc

N_CLS_K = 1000
SEQ_K = 77
N_CTX_K = 16
CTX_DIM_K = 512
CHUNK = 64
N_CHUNKS = 16
LAST_START = N_CLS_K - CHUNK
N_TASKS = SEQ_K * N_CHUNKS

NC, NS = 2, 16
NW = NC * NS
BASE_T = N_TASKS // NW
EXTRA_T = N_TASKS - BASE_T * NW
ROUNDS = BASE_T + 1
assert ROUNDS % 3 == 0


def _make_kernel():
    mesh = plsc.VectorSubcoreMesh(core_axis_name="c", subcore_axis_name="s")

    @functools.partial(
        pl.kernel,
        mesh=mesh,
        out_type=jax.ShapeDtypeStruct((SEQ_K, N_CLS_K, CTX_DIM_K), jnp.float32),
        scratch_types=[
            pltpu.VMEM((CHUNK, CTX_DIM_K), jnp.float32),
            pltpu.VMEM((CHUNK, CTX_DIM_K), jnp.float32),
            pltpu.VMEM((CHUNK, CTX_DIM_K), jnp.float32),
            pltpu.VMEM((CHUNK,), jnp.int32),
            pltpu.VMEM((CHUNK,), jnp.int32),
            pltpu.VMEM((CHUNK,), jnp.int32),
            pltpu.SemaphoreType.DMA,
            pltpu.SemaphoreType.DMA,
            pltpu.SemaphoreType.DMA,
            pltpu.SemaphoreType.DMA,
            pltpu.SemaphoreType.DMA,
            pltpu.SemaphoreType.DMA,
            pltpu.SemaphoreType.DMA,
            pltpu.SemaphoreType.DMA,
            pltpu.SemaphoreType.DMA,
        ],
    )
    def sc_kernel(
        idx_hbm, table_hbm, ctxb_hbm, out_hbm,
        buf0, buf1, buf2, ib0, ib1, ib2,
        si0, si1, si2, sin0, sin1, sin2, sw0, sw1, sw2,
    ):
        bufs, ibufs = (buf0, buf1, buf2), (ib0, ib1, ib2)
        sidx, sin, sw = (si0, si1, si2), (sin0, sin1, sin2), (sw0, sw1, sw2)
        wid = lax.axis_index("s") * NC + lax.axis_index("c")
        n_t = jnp.where(wid < EXTRA_T, BASE_T + 1, BASE_T)

        def params(t_local):
            t = wid + NW * lax.rem(t_local, n_t)
            s = t // N_CHUNKS
            ch = lax.rem(t, N_CHUNKS)
            c0 = pl.multiple_of(
                jnp.where(ch == N_CHUNKS - 1, LAST_START, ch * CHUNK), 8
            )
            is_ctx = jnp.logical_and(s >= 1, s < 1 + N_CTX_K)
            row = jnp.where(is_ctx, 0, jnp.where(s == 0, 0, s - N_CTX_K))
            ioff = pl.multiple_of(row * N_CLS_K + c0, 8)
            return s, c0, is_ctx, ioff

        def stage_idx(t_local, b):
            _, _, _, ioff = params(t_local)
            pltpu.async_copy(
                idx_hbm.at[pl.ds(ioff, CHUNK)], ibufs[b], sidx[b]
            )

        def wait_idx(b):
            pltpu.make_async_copy(
                idx_hbm.at[pl.ds(0, CHUNK)], ibufs[b], sidx[b]
            ).wait()

        def issue_main(t_local, b):
            s, _, is_ctx, _ = params(t_local)

            @pl.when(is_ctx)
            def _():
                pltpu.async_copy(
                    ctxb_hbm.at[s - 1], bufs[b].at[pl.ds(0, 8)], sin[b]
                )

            @pl.when(jnp.logical_not(is_ctx))
            def _():
                pltpu.async_copy(table_hbm.at[ibufs[b]], bufs[b], sin[b])

        def wait_main(t_local, b):
            _, _, is_ctx, _ = params(t_local)

            @pl.when(is_ctx)
            def _():
                pltpu.make_async_copy(
                    ctxb_hbm.at[0], bufs[b].at[pl.ds(0, 8)], sin[b]
                ).wait()

            @pl.when(jnp.logical_not(is_ctx))
            def _():
                pltpu.make_async_copy(
                    table_hbm.at[pl.ds(0, CHUNK)], bufs[b], sin[b]
                ).wait()

        def drain_write(b):
            pltpu.make_async_copy(
                bufs[b], out_hbm.at[0, pl.ds(0, CHUNK)], sw[b]
            ).wait()

        stage_idx(0, 0)
        stage_idx(1, 1)
        wait_idx(0)
        issue_main(0, 0)

        def tripple(tt, carry):
            for b in (0, 1, 2):
                t = tt * 3 + b
                b1, b2 = (b + 1) % 3, (b + 2) % 3
                s, c0, is_ctx, _ = params(t)
                wait_main(t, b)

                @pl.when(is_ctx)
                def _():
                    for i in range(8):
                        pltpu.async_copy(
                            bufs[b].at[pl.ds(0, 8)],
                            out_hbm.at[s, pl.ds(c0 + 8 * i, 8)],
                            sw[b],
                        )

                @pl.when(jnp.logical_not(is_ctx))
                def _():
                    pltpu.async_copy(
                        bufs[b], out_hbm.at[s, pl.ds(c0, CHUNK)], sw[b]
                    )

                @pl.when(t + 2 < ROUNDS)
                def _():
                    stage_idx(t + 2, b2)

                @pl.when(t + 1 < ROUNDS)
                def _():
                    wait_idx(b1)

                    @pl.when(t >= 2)
                    def _():
                        drain_write(b1)

                    issue_main(t + 1, b1)
            return carry

        lax.fori_loop(0, ROUNDS // 3, tripple, 0)
        drain_write(0)
        drain_write(1)
        drain_write(2)

    return sc_kernel


_SC_KERNEL = _make_kernel()


@jax.jit
def kernel(tokenized_prompts, token_embedding, ctx):
    cols = jnp.concatenate(
        [tokenized_prompts[:, :1], tokenized_prompts[:, 1 + N_CTX_K :]], axis=1
    )
    idx = cols.T.reshape(-1)
    ctxb = jnp.broadcast_to(ctx[:, None, :], (N_CTX_K, 8, CTX_DIM_K))
    out_t = _SC_KERNEL(idx, token_embedding, ctxb)
    return jnp.transpose(out_t, (1, 0, 2))


# speedup vs baseline: 8.9838x; 1.0238x over previous
import functools

import jax
import jax.numpy as jnp
from jax import lax
from jax.experimental import pallas as pl
from jax.experimental.pallas import tpu as pltpu
from jax.experimental.pallas import tpu_sc as plsc

N_CLS_K = 1000
SEQ_K = 77
N_CTX_K = 16
CTX_DIM_K = 512
CHUNK = 48
N_CHUNKS = 21
LAST_START = N_CLS_K - CHUNK
N_TASKS = SEQ_K * N_CHUNKS
NBUF = 4

NC, NS = 2, 16
NW = NC * NS
BASE_T = N_TASKS // NW
EXTRA_T = N_TASKS - BASE_T * NW
ROUNDS = 52
assert ROUNDS % NBUF == 0 and ROUNDS >= BASE_T + 1


def _make_kernel():
    mesh = plsc.VectorSubcoreMesh(core_axis_name="c", subcore_axis_name="s")

    @functools.partial(
        pl.kernel,
        mesh=mesh,
        out_type=jax.ShapeDtypeStruct((SEQ_K, N_CLS_K, CTX_DIM_K), jnp.float32),
        scratch_types=(
            [pltpu.VMEM((CHUNK, CTX_DIM_K), jnp.float32) for _ in range(NBUF)]
            + [pltpu.VMEM((CHUNK,), jnp.int32) for _ in range(NBUF)]
            + [pltpu.SemaphoreType.DMA for _ in range(3 * NBUF)]
        ),
    )
    def sc_kernel(idx_hbm, table_hbm, ctxb_hbm, out_hbm, *scratch):
        bufs = scratch[:NBUF]
        ibufs = scratch[NBUF : 2 * NBUF]
        sidx = scratch[2 * NBUF : 3 * NBUF]
        sin = scratch[3 * NBUF : 4 * NBUF]
        sw = scratch[4 * NBUF : 5 * NBUF]
        wid = lax.axis_index("s") * NC + lax.axis_index("c")
        n_t = jnp.where(wid < EXTRA_T, BASE_T + 1, BASE_T)

        def params(t_local):
            t = wid + NW * lax.rem(t_local, n_t)
            s = t // N_CHUNKS
            ch = lax.rem(t, N_CHUNKS)
            c0 = pl.multiple_of(
                jnp.where(ch == N_CHUNKS - 1, LAST_START, ch * CHUNK), 8
            )
            is_ctx = jnp.logical_and(s >= 1, s < 1 + N_CTX_K)
            row = jnp.where(jnp.logical_or(is_ctx, s == 0), 0, s - N_CTX_K)
            ioff = pl.multiple_of(row * N_CLS_K + c0, 8)
            return s, c0, is_ctx, ioff

        def stage_idx(t_local, b):
            _, _, _, ioff = params(t_local)
            pltpu.async_copy(idx_hbm.at[pl.ds(ioff, CHUNK)], ibufs[b], sidx[b])

        def wait_idx(b):
            pltpu.make_async_copy(
                idx_hbm.at[pl.ds(0, CHUNK)], ibufs[b], sidx[b]
            ).wait()

        def issue_main(t_local, b):
            s, _, is_ctx, _ = params(t_local)

            @pl.when(is_ctx)
            def _():
                pltpu.async_copy(
                    ctxb_hbm.at[s - 1], bufs[b].at[pl.ds(0, 8)], sin[b]
                )

            @pl.when(jnp.logical_not(is_ctx))
            def _():
                pltpu.async_copy(table_hbm.at[ibufs[b]], bufs[b], sin[b])

        def wait_main(t_local, b):
            _, _, is_ctx, _ = params(t_local)

            @pl.when(is_ctx)
            def _():
                pltpu.make_async_copy(
                    ctxb_hbm.at[0], bufs[b].at[pl.ds(0, 8)], sin[b]
                ).wait()

            @pl.when(jnp.logical_not(is_ctx))
            def _():
                pltpu.make_async_copy(
                    table_hbm.at[pl.ds(0, CHUNK)], bufs[b], sin[b]
                ).wait()

        def issue_write(t_local, b):
            s, c0, is_ctx, _ = params(t_local)

            @pl.when(is_ctx)
            def _():
                for i in range(CHUNK // 8):
                    pltpu.async_copy(
                        bufs[b].at[pl.ds(0, 8)],
                        out_hbm.at[s, pl.ds(c0 + 8 * i, 8)],
                        sw[b],
                    )

            @pl.when(jnp.logical_not(is_ctx))
            def _():
                pltpu.async_copy(
                    bufs[b], out_hbm.at[s, pl.ds(c0, CHUNK)], sw[b]
                )

        def drain_write(b):
            pltpu.make_async_copy(
                bufs[b], out_hbm.at[0, pl.ds(0, CHUNK)], sw[b]
            ).wait()

        stage_idx(0, 0)
        stage_idx(1, 1)
        stage_idx(2, 2)
        wait_idx(0)
        issue_main(0, 0)
        wait_idx(1)
        issue_main(1, 1)

        def quad(tt, carry):
            for b in range(NBUF):
                t = tt * NBUF + b
                b2, b3 = (b + 2) % NBUF, (b + 3) % NBUF
                wait_main(t, b)
                issue_write(t, b)

                @pl.when(t + 3 < ROUNDS)
                def _():
                    stage_idx(t + 3, b3)

                @pl.when(t + 2 < ROUNDS)
                def _():
                    wait_idx(b2)

                    @pl.when(t >= 2)
                    def _():
                        drain_write(b2)

                    issue_main(t + 2, b2)
            return carry

        lax.fori_loop(0, ROUNDS // NBUF, quad, 0)
        for b in range(NBUF):
            drain_write(b)

    return sc_kernel


_SC_KERNEL = _make_kernel()


@jax.jit
def kernel(tokenized_prompts, token_embedding, ctx):
    cols = jnp.concatenate(
        [tokenized_prompts[:, :1], tokenized_prompts[:, 1 + N_CTX_K :]], axis=1
    )
    idx = cols.T.reshape(-1)
    ctxb = jnp.broadcast_to(ctx[:, None, :], (N_CTX_K, 8, CTX_DIM_K))
    out_t = _SC_KERNEL(idx, token_embedding, ctxb)
    return jnp.transpose(out_t, (1, 0, 2))
